# baseline (speedup 1.0000x reference)
import functools

import jax
import jax.numpy as jnp
from jax import lax
from jax.experimental import pallas as pl
from jax.experimental.pallas import tpu as pltpu
from jax.experimental.pallas import tpu_sc as plsc

D = 64
MARGIN = 0.1
BN_EPS = 1e-5

NW = 32
NC = 2
CH = 512
HP = CH // 2

NCOLS = 9
NRELCOLS = 2
NCH = NCOLS + NRELCOLS


def _sc_gather(idx_nat, idx_rel, class_embed, class_rad, rel_embed, b):
    nblk = b // CH
    assert nblk == NW

    mesh = plsc.VectorSubcoreMesh(core_axis_name="c", subcore_axis_name="s")

    def body(idxn_hbm, ridx_hbm, table_hbm, rad_hbm, rel_hbm,
             emb_out, rad_out, rel_out,
             idxn_v, buf0, buf1, buf2, rbuf,
             gsem0, gsem1, gsem2, wsem0, wsem1, wsem2, rsem):
        wid = lax.axis_index("s") * NC + lax.axis_index("c")
        bufs = (buf0, buf1, buf2)
        gsems = (gsem0, gsem1, gsem2)
        wsems = (wsem0, wsem1, wsem2)

        for t in range(NCOLS):
            pltpu.sync_copy(idxn_hbm.at[t, wid],
                            idxn_v.at[pl.ds(t * CH, CH)])
        for t in range(NRELCOLS):
            pltpu.sync_copy(ridx_hbm.at[t, wid],
                            idxn_v.at[pl.ds((NCOLS + t) * CH, CH)])

        rad_cp = pltpu.async_copy(
            rad_hbm.at[idxn_v.at[pl.ds(0, NCOLS * CH)]], rbuf, rsem)

        def gather(t):
            src = table_hbm if t < NCOLS else rel_hbm
            return pltpu.async_copy(
                src.at[idxn_v.at[pl.ds(t * CH, CH)]], bufs[t % 3],
                gsems[t % 3])

        def write(t):
            bf = bufs[t % 3]
            if t < NCOLS:
                dst = emb_out.at[t, pl.ds(wid * HP, HP)]
            else:
                dst = rel_out.at[t - NCOLS, pl.ds(wid * HP, HP)]
            lo = pltpu.async_copy(bf.at[pl.ds(0, HP)],
                                  dst.at[:, pl.ds(0, D)], wsems[t % 3])
            hi = pltpu.async_copy(bf.at[pl.ds(HP, HP)],
                                  dst.at[:, pl.ds(D, D)], wsems[t % 3])
            return (lo, hi)

        gcp = {0: gather(0), 1: gather(1)}
        wcp = {}
        for t in range(NCH):
            gcp[t].wait()
            wcp[t] = write(t)
            if t + 2 < NCH:
                if t - 1 >= 0:
                    wcp[t - 1][0].wait()
                    wcp[t - 1][1].wait()
                gcp[t + 2] = gather(t + 2)
        wcp[NCH - 2][0].wait()
        wcp[NCH - 2][1].wait()
        wcp[NCH - 1][0].wait()
        wcp[NCH - 1][1].wait()

        rad_cp.wait()
        for t in range(NCOLS):
            pltpu.sync_copy(rbuf.at[pl.ds(t * CH, CH)],
                            rad_out.at[pl.ds(t * NW * CH + wid * CH, CH)])

    k = pl.kernel(
        body,
        out_type=(
            jax.ShapeDtypeStruct((NCOLS, b // 2, 2 * D), jnp.float32),
            jax.ShapeDtypeStruct((NCOLS * b,), jnp.float32),
            jax.ShapeDtypeStruct((NRELCOLS, b // 2, 2 * D), jnp.float32),
        ),
        mesh=mesh,
        compiler_params=pltpu.CompilerParams(use_tc_tiling_on_sc=False),
        scratch_types=[
            pltpu.VMEM((NCH * CH,), jnp.int32),
            pltpu.VMEM((CH, D), jnp.float32),
            pltpu.VMEM((CH, D), jnp.float32),
            pltpu.VMEM((CH, D), jnp.float32),
            pltpu.VMEM((NCOLS * CH,), jnp.float32),
            pltpu.SemaphoreType.DMA,
            pltpu.SemaphoreType.DMA,
            pltpu.SemaphoreType.DMA,
            pltpu.SemaphoreType.DMA,
            pltpu.SemaphoreType.DMA,
            pltpu.SemaphoreType.DMA,
            pltpu.SemaphoreType.DMA,
        ],
    )
    return k(idx_nat, idx_rel, class_embed, _depad_rad(class_rad), rel_embed)


def _depad_body(src_ref, out_ref):
    out_ref[...] = jnp.sum(src_ref[...], axis=1)


def _depad_rad(class_rad):
    n = class_rad.shape[0]
    blk = 8192
    return pl.pallas_call(
        _depad_body,
        grid=(pl.cdiv(n, blk),),
        in_specs=[pl.BlockSpec((blk, 1), lambda i: (i, 0))],
        out_specs=pl.BlockSpec((blk,), lambda i: (i,)),
        out_shape=jax.ShapeDtypeStruct((n,), jnp.float32),
    )(class_rad)


def _stats_body(emb_ref, sum_ref, sq_ref):
    i = pl.program_id(0)
    x = emb_ref[...]
    s = jnp.sum(x, axis=1)
    q = jnp.sum(x * x, axis=1)

    @pl.when(i == 0)
    def _():
        sum_ref[...] = s
        sq_ref[...] = q

    @pl.when(i > 0)
    def _():
        sum_ref[...] += s
        sq_ref[...] += q


def _loss_body(emb_ref, rad_ref, rel_ref, sum_ref, sq_ref, gamma_ref,
               beta_ref, out_ref, *, batch):
    i = pl.program_id(0)
    inv_b = jnp.float32(1.0 / batch)
    s64 = sum_ref[:, :D] + sum_ref[:, D:]
    q64 = sq_ref[:, :D] + sq_ref[:, D:]
    mean = s64 * inv_b
    var = q64 * inv_b - mean * mean
    scale = gamma_ref[:, :D] * lax.rsqrt(var + BN_EPS)
    shift = beta_ref[:, :D] - mean * scale
    scale2 = jnp.concatenate([scale, scale], axis=-1)
    shift2 = jnp.concatenate([shift, shift], axis=-1)

    x = emb_ref[...]
    h = x * scale2[:, None, :] + shift2[:, None, :]
    r = jnp.abs(rad_ref[...])[:, 0]
    rel = rel_ref[...]

    lane = lax.broadcasted_iota(jnp.int32, (HP, 2 * D), 1)
    mlo = lane < D

    def norms(v):
        d2 = v * v
        qlo = jnp.sum(jnp.where(mlo, d2, 0.0), axis=-1)
        qhi = jnp.sum(jnp.where(mlo, 0.0, d2), axis=-1)
        tiny = jnp.float32(1e-12)
        return (jnp.sqrt(qlo + tiny).reshape(2, D * 2),
                jnp.sqrt(qhi + tiny).reshape(2, D * 2))

    relu = jax.nn.relu
    m = jnp.float32(MARGIN)

    def rlo(g):
        return r[g, 0:2, :]

    def rhi(g):
        return r[g, 2:4, :]

    total = jnp.float32(0.0)
    nlo, nhi = norms(h[0] - h[1])
    total += jnp.sum(relu(nlo + rlo(0) - rlo(1) - m))
    total += jnp.sum(relu(nhi + rhi(0) - rhi(1) - m))
    nlo, nhi = norms(h[3] - h[2])
    total += jnp.sum(relu(nlo - (rlo(2) + rlo(3)) - m))
    total += jnp.sum(relu(nhi - (rhi(2) + rhi(3)) - m))
    nlo, nhi = norms(h[4] - h[2])
    total += jnp.sum(relu(nlo - rlo(2) - m))
    total += jnp.sum(relu(nhi - rhi(2) - m))
    nlo, nhi = norms(h[4] - h[3])
    total += jnp.sum(relu(nlo - rlo(3) - m))
    total += jnp.sum(relu(nhi - rhi(3) - m))
    nlo, nhi = norms(h[5] + rel[0] - h[6])
    total += jnp.sum(relu(nlo + rlo(5) - rlo(6) - m))
    total += jnp.sum(relu(nhi + rhi(5) - rhi(6) - m))
    total += jnp.sum(relu(rlo(5) + rlo(6) - nlo + m))
    total += jnp.sum(relu(rhi(5) + rhi(6) - nhi + m))
    nlo, nhi = norms(h[7] - rel[1] - h[8])
    total += jnp.sum(relu(nlo - rlo(7) - rlo(8) - m))
    total += jnp.sum(relu(nhi - rhi(7) - rhi(8) - m))

    total = total.reshape(1, 1)

    @pl.when(i == 0)
    def _():
        out_ref[...] = total

    @pl.when(i > 0)
    def _():
        out_ref[...] += total

    @pl.when(i == pl.num_programs(0) - 1)
    def _():
        out_ref[...] *= inv_b


def kernel(gci0, gci1, gci2, gci3, class_embed, class_rad, rel_embed,
           bn_gamma, bn_beta):
    b = gci0.shape[0]
    cols = jnp.stack(
        [gci0[:, 0], gci0[:, 1],
         gci1[:, 0], gci1[:, 1], gci1[:, 2],
         gci2[:, 0], gci2[:, 2],
         gci3[:, 1], gci3[:, 2]], axis=0)
    relcols = jnp.stack([gci2[:, 1], gci3[:, 0]], axis=0)
    idx_nat = cols.reshape(NCOLS, b // CH, CH)
    idx_rel = relcols.reshape(NRELCOLS, b // CH, CH)

    emb4, rad_all, rel4 = _sc_gather(
        idx_nat, idx_rel, class_embed, class_rad, rel_embed, b)

    rad3 = rad_all.reshape(NCOLS, b // CH, 4, 2 * D)
    g2 = jnp.concatenate([bn_gamma, bn_gamma]).reshape(1, 2 * D)
    b2 = jnp.concatenate([bn_beta, bn_beta]).reshape(1, 2 * D)

    ng = b // CH

    sums, sqs = pl.pallas_call(
        _stats_body,
        grid=(ng,),
        in_specs=[pl.BlockSpec((NCOLS, HP, 2 * D), lambda i: (0, i, 0))],
        out_specs=(pl.BlockSpec((NCOLS, 2 * D), lambda i: (0, 0)),
                   pl.BlockSpec((NCOLS, 2 * D), lambda i: (0, 0))),
        out_shape=(jax.ShapeDtypeStruct((NCOLS, 2 * D), jnp.float32),
                   jax.ShapeDtypeStruct((NCOLS, 2 * D), jnp.float32)),
    )(emb4)

    loss = pl.pallas_call(
        functools.partial(_loss_body, batch=b),
        grid=(ng,),
        in_specs=[
            pl.BlockSpec((NCOLS, HP, 2 * D), lambda i: (0, i, 0)),
            pl.BlockSpec((NCOLS, 1, 4, 2 * D), lambda i: (0, i, 0, 0)),
            pl.BlockSpec((NRELCOLS, HP, 2 * D), lambda i: (0, i, 0)),
            pl.BlockSpec((NCOLS, 2 * D), lambda i: (0, 0)),
            pl.BlockSpec((NCOLS, 2 * D), lambda i: (0, 0)),
            pl.BlockSpec((1, 2 * D), lambda i: (0, 0)),
            pl.BlockSpec((1, 2 * D), lambda i: (0, 0)),
        ],
        out_specs=pl.BlockSpec((1, 1), lambda i: (0, 0)),
        out_shape=jax.ShapeDtypeStruct((1, 1), jnp.float32),
    )(emb4, rad3, rel4, sums, sqs, g2, b2)

    return loss[0, 0]


# speedup vs baseline: 1.6817x; 1.6817x over previous
import functools

import jax
import jax.numpy as jnp
from jax import lax
from jax.experimental import pallas as pl
from jax.experimental.pallas import tpu as pltpu
from jax.experimental.pallas import tpu_sc as plsc

D = 64
MARGIN = 0.1
BN_EPS = 1e-5

NW = 32
NC = 2
CH = 512
HP = CH // 2

NCOLS = 9
NRELCOLS = 2
NCH = NCOLS + NRELCOLS


def _sc_gather(idx_nat, idx_rel, class_embed, class_rad, rel_embed, b):
    nblk = b // CH
    assert nblk == NW

    mesh = plsc.VectorSubcoreMesh(core_axis_name="c", subcore_axis_name="s")

    def body(idxn_hbm, ridx_hbm, table_hbm, rad_hbm, rel_hbm,
             emb_out, rad_out, rel_out,
             idxn_v, buf0, buf1, buf2, rbuf,
             gsem0, gsem1, gsem2, wsem0, wsem1, wsem2, rsem):
        wid = lax.axis_index("s") * NC + lax.axis_index("c")
        bufs = (buf0, buf1, buf2)
        gsems = (gsem0, gsem1, gsem2)
        wsems = (wsem0, wsem1, wsem2)

        for t in range(NCOLS):
            pltpu.sync_copy(idxn_hbm.at[t, wid],
                            idxn_v.at[pl.ds(t * CH, CH)])
        for t in range(NRELCOLS):
            pltpu.sync_copy(ridx_hbm.at[t, wid],
                            idxn_v.at[pl.ds((NCOLS + t) * CH, CH)])

        rad_cp = pltpu.async_copy(
            rad_hbm.at[idxn_v.at[pl.ds(0, NCOLS * CH)]], rbuf, rsem)

        def gather(t):
            src = table_hbm if t < NCOLS else rel_hbm
            return pltpu.async_copy(
                src.at[idxn_v.at[pl.ds(t * CH, CH)]], bufs[t % 3],
                gsems[t % 3])

        def write(t):
            bf = bufs[t % 3]
            if t < NCOLS:
                dst = emb_out.at[t, pl.ds(wid * HP, HP)]
            else:
                dst = rel_out.at[t - NCOLS, pl.ds(wid * HP, HP)]
            lo = pltpu.async_copy(bf.at[pl.ds(0, HP)],
                                  dst.at[:, pl.ds(0, D)], wsems[t % 3])
            hi = pltpu.async_copy(bf.at[pl.ds(HP, HP)],
                                  dst.at[:, pl.ds(D, D)], wsems[t % 3])
            return (lo, hi)

        gcp = {0: gather(0), 1: gather(1)}
        wcp = {}
        for t in range(NCH):
            gcp[t].wait()
            wcp[t] = write(t)
            if t + 2 < NCH:
                if t - 1 >= 0:
                    wcp[t - 1][0].wait()
                    wcp[t - 1][1].wait()
                gcp[t + 2] = gather(t + 2)
        wcp[NCH - 2][0].wait()
        wcp[NCH - 2][1].wait()
        wcp[NCH - 1][0].wait()
        wcp[NCH - 1][1].wait()

        rad_cp.wait()
        for t in range(NCOLS):
            pltpu.sync_copy(rbuf.at[pl.ds(t * CH, CH)],
                            rad_out.at[pl.ds(t * NW * CH + wid * CH, CH)])

    k = pl.kernel(
        body,
        out_type=(
            jax.ShapeDtypeStruct((NCOLS, b // 2, 2 * D), jnp.float32),
            jax.ShapeDtypeStruct((NCOLS * b,), jnp.float32),
            jax.ShapeDtypeStruct((NRELCOLS, b // 2, 2 * D), jnp.float32),
        ),
        mesh=mesh,
        compiler_params=pltpu.CompilerParams(use_tc_tiling_on_sc=False),
        scratch_types=[
            pltpu.VMEM((NCH * CH,), jnp.int32),
            pltpu.VMEM((CH, D), jnp.float32),
            pltpu.VMEM((CH, D), jnp.float32),
            pltpu.VMEM((CH, D), jnp.float32),
            pltpu.VMEM((NCOLS * CH,), jnp.float32),
            pltpu.SemaphoreType.DMA,
            pltpu.SemaphoreType.DMA,
            pltpu.SemaphoreType.DMA,
            pltpu.SemaphoreType.DMA,
            pltpu.SemaphoreType.DMA,
            pltpu.SemaphoreType.DMA,
            pltpu.SemaphoreType.DMA,
        ],
    )
    return k(idx_nat, idx_rel, class_embed,
             jnp.transpose(class_rad).reshape(-1), rel_embed)


def _stats_body(emb_ref, sum_ref, sq_ref):
    i = pl.program_id(0)
    x = emb_ref[...]
    s = jnp.sum(x, axis=1)
    q = jnp.sum(x * x, axis=1)

    @pl.when(i == 0)
    def _():
        sum_ref[...] = s
        sq_ref[...] = q

    @pl.when(i > 0)
    def _():
        sum_ref[...] += s
        sq_ref[...] += q


def _loss_body(emb_ref, rad_ref, rel_ref, sum_ref, sq_ref, gamma_ref,
               beta_ref, out_ref, *, batch):
    i = pl.program_id(0)
    inv_b = jnp.float32(1.0 / batch)
    s64 = sum_ref[:, :D] + sum_ref[:, D:]
    q64 = sq_ref[:, :D] + sq_ref[:, D:]
    mean = s64 * inv_b
    var = q64 * inv_b - mean * mean
    scale = gamma_ref[:, :D] * lax.rsqrt(var + BN_EPS)
    shift = beta_ref[:, :D] - mean * scale
    scale2 = jnp.concatenate([scale, scale], axis=-1)
    shift2 = jnp.concatenate([shift, shift], axis=-1)

    x = emb_ref[...]
    h = x * scale2[:, None, :] + shift2[:, None, :]
    r = jnp.abs(rad_ref[...])[:, 0]
    rel = rel_ref[...]

    lane = lax.broadcasted_iota(jnp.int32, (HP, 2 * D), 1)
    mlo = lane < D

    def norms(v):
        d2 = v * v
        qlo = jnp.sum(jnp.where(mlo, d2, 0.0), axis=-1)
        qhi = jnp.sum(jnp.where(mlo, 0.0, d2), axis=-1)
        tiny = jnp.float32(1e-12)
        return (jnp.sqrt(qlo + tiny).reshape(2, D * 2),
                jnp.sqrt(qhi + tiny).reshape(2, D * 2))

    relu = jax.nn.relu
    m = jnp.float32(MARGIN)

    def rlo(g):
        return r[g, 0:2, :]

    def rhi(g):
        return r[g, 2:4, :]

    total = jnp.float32(0.0)
    nlo, nhi = norms(h[0] - h[1])
    total += jnp.sum(relu(nlo + rlo(0) - rlo(1) - m))
    total += jnp.sum(relu(nhi + rhi(0) - rhi(1) - m))
    nlo, nhi = norms(h[3] - h[2])
    total += jnp.sum(relu(nlo - (rlo(2) + rlo(3)) - m))
    total += jnp.sum(relu(nhi - (rhi(2) + rhi(3)) - m))
    nlo, nhi = norms(h[4] - h[2])
    total += jnp.sum(relu(nlo - rlo(2) - m))
    total += jnp.sum(relu(nhi - rhi(2) - m))
    nlo, nhi = norms(h[4] - h[3])
    total += jnp.sum(relu(nlo - rlo(3) - m))
    total += jnp.sum(relu(nhi - rhi(3) - m))
    nlo, nhi = norms(h[5] + rel[0] - h[6])
    total += jnp.sum(relu(nlo + rlo(5) - rlo(6) - m))
    total += jnp.sum(relu(nhi + rhi(5) - rhi(6) - m))
    total += jnp.sum(relu(rlo(5) + rlo(6) - nlo + m))
    total += jnp.sum(relu(rhi(5) + rhi(6) - nhi + m))
    nlo, nhi = norms(h[7] - rel[1] - h[8])
    total += jnp.sum(relu(nlo - rlo(7) - rlo(8) - m))
    total += jnp.sum(relu(nhi - rhi(7) - rhi(8) - m))

    total = total.reshape(1, 1)

    @pl.when(i == 0)
    def _():
        out_ref[...] = total

    @pl.when(i > 0)
    def _():
        out_ref[...] += total

    @pl.when(i == pl.num_programs(0) - 1)
    def _():
        out_ref[...] *= inv_b


def kernel(gci0, gci1, gci2, gci3, class_embed, class_rad, rel_embed,
           bn_gamma, bn_beta):
    b = gci0.shape[0]
    cols = jnp.stack(
        [gci0[:, 0], gci0[:, 1],
         gci1[:, 0], gci1[:, 1], gci1[:, 2],
         gci2[:, 0], gci2[:, 2],
         gci3[:, 1], gci3[:, 2]], axis=0)
    relcols = jnp.stack([gci2[:, 1], gci3[:, 0]], axis=0)
    idx_nat = cols.reshape(NCOLS, b // CH, CH)
    idx_rel = relcols.reshape(NRELCOLS, b // CH, CH)

    emb4, rad_all, rel4 = _sc_gather(
        idx_nat, idx_rel, class_embed, class_rad, rel_embed, b)

    rad3 = rad_all.reshape(NCOLS, b // CH, 4, 2 * D)
    g2 = jnp.concatenate([bn_gamma, bn_gamma]).reshape(1, 2 * D)
    b2 = jnp.concatenate([bn_beta, bn_beta]).reshape(1, 2 * D)

    ng = b // CH

    sums, sqs = pl.pallas_call(
        _stats_body,
        grid=(ng,),
        in_specs=[pl.BlockSpec((NCOLS, HP, 2 * D), lambda i: (0, i, 0))],
        out_specs=(pl.BlockSpec((NCOLS, 2 * D), lambda i: (0, 0)),
                   pl.BlockSpec((NCOLS, 2 * D), lambda i: (0, 0))),
        out_shape=(jax.ShapeDtypeStruct((NCOLS, 2 * D), jnp.float32),
                   jax.ShapeDtypeStruct((NCOLS, 2 * D), jnp.float32)),
    )(emb4)

    loss = pl.pallas_call(
        functools.partial(_loss_body, batch=b),
        grid=(ng,),
        in_specs=[
            pl.BlockSpec((NCOLS, HP, 2 * D), lambda i: (0, i, 0)),
            pl.BlockSpec((NCOLS, 1, 4, 2 * D), lambda i: (0, i, 0, 0)),
            pl.BlockSpec((NRELCOLS, HP, 2 * D), lambda i: (0, i, 0)),
            pl.BlockSpec((NCOLS, 2 * D), lambda i: (0, 0)),
            pl.BlockSpec((NCOLS, 2 * D), lambda i: (0, 0)),
            pl.BlockSpec((1, 2 * D), lambda i: (0, 0)),
            pl.BlockSpec((1, 2 * D), lambda i: (0, 0)),
        ],
        out_specs=pl.BlockSpec((1, 1), lambda i: (0, 0)),
        out_shape=jax.ShapeDtypeStruct((1, 1), jnp.float32),
    )(emb4, rad3, rel4, sums, sqs, g2, b2)

    return loss[0, 0]


# speedup vs baseline: 1.7832x; 1.0603x over previous
import functools

import jax
import jax.numpy as jnp
from jax import lax
from jax.experimental import pallas as pl
from jax.experimental.pallas import tpu as pltpu
from jax.experimental.pallas import tpu_sc as plsc

D = 64
MARGIN = 0.1
BN_EPS = 1e-5

NW = 32
NC = 2
CH = 512
HP = CH // 2

NCOLS = 9
NRELCOLS = 2
NCH = NCOLS + NRELCOLS


def _sc_gather(idx_nat, idx_rel, class_embed, class_rad, rel_embed, b):
    nblk = b // CH
    assert nblk == NW

    mesh = plsc.VectorSubcoreMesh(core_axis_name="c", subcore_axis_name="s")

    def body(idxn_hbm, ridx_hbm, table_hbm, rad_hbm, rel_hbm,
             emb_out, rad_out, rel_out,
             idxn_v, buf0, buf1, buf2, rbuf,
             gsem0, gsem1, gsem2, wsem0, wsem1, wsem2, rsem):
        wid = lax.axis_index("s") * NC + lax.axis_index("c")
        bufs = (buf0, buf1, buf2)
        gsems = (gsem0, gsem1, gsem2)
        wsems = (wsem0, wsem1, wsem2)

        for t in range(NCOLS):
            pltpu.sync_copy(idxn_hbm.at[t, wid],
                            idxn_v.at[pl.ds(t * CH, CH)])
        for t in range(NRELCOLS):
            pltpu.sync_copy(ridx_hbm.at[t, wid],
                            idxn_v.at[pl.ds((NCOLS + t) * CH, CH)])

        rad_cp = pltpu.async_copy(
            rad_hbm.at[idxn_v.at[pl.ds(0, NCOLS * CH)]], rbuf, rsem)

        nch2 = 2 * NCH

        def gather(t):
            src = table_hbm if t < 2 * NCOLS else rel_hbm
            return pltpu.async_copy(
                src.at[idxn_v.at[pl.ds(t * HP, HP)]], bufs[t % 3],
                gsems[t % 3])

        def write(t):
            col, half = t // 2, t % 2
            if col < NCOLS:
                dst = emb_out.at[col, pl.ds(wid * HP, HP)]
            else:
                dst = rel_out.at[col - NCOLS, pl.ds(wid * HP, HP)]
            return pltpu.async_copy(
                bufs[t % 3].at[:, pl.ds(0, D)],
                dst.at[:, pl.ds(half * D, D)], wsems[t % 3])

        gcp = {0: gather(0), 1: gather(1)}
        wcp = {}
        for t in range(nch2):
            gcp[t].wait()
            wcp[t] = write(t)
            if t + 2 < nch2:
                if t - 1 >= 0:
                    wcp[t - 1].wait()
                gcp[t + 2] = gather(t + 2)
        wcp[nch2 - 2].wait()
        wcp[nch2 - 1].wait()

        rad_cp.wait()
        for t in range(NCOLS):
            pltpu.sync_copy(rbuf.at[pl.ds(t * CH, CH)],
                            rad_out.at[pl.ds(t * NW * CH + wid * CH, CH)])

    k = pl.kernel(
        body,
        out_type=(
            jax.ShapeDtypeStruct((NCOLS, b // 2, 2 * D), jnp.float32),
            jax.ShapeDtypeStruct((NCOLS * b,), jnp.float32),
            jax.ShapeDtypeStruct((NRELCOLS, b // 2, 2 * D), jnp.float32),
        ),
        mesh=mesh,
        compiler_params=pltpu.CompilerParams(use_tc_tiling_on_sc=False),
        scratch_types=[
            pltpu.VMEM((NCH * CH,), jnp.int32),
            pltpu.VMEM((HP, 2 * D), jnp.float32),
            pltpu.VMEM((HP, 2 * D), jnp.float32),
            pltpu.VMEM((HP, 2 * D), jnp.float32),
            pltpu.VMEM((NCOLS * CH,), jnp.float32),
            pltpu.SemaphoreType.DMA,
            pltpu.SemaphoreType.DMA,
            pltpu.SemaphoreType.DMA,
            pltpu.SemaphoreType.DMA,
            pltpu.SemaphoreType.DMA,
            pltpu.SemaphoreType.DMA,
            pltpu.SemaphoreType.DMA,
        ],
    )
    return k(idx_nat, idx_rel,
             jnp.pad(class_embed, ((0, 0), (0, D))),
             jnp.transpose(class_rad).reshape(-1),
             jnp.pad(rel_embed, ((0, 0), (0, D))))


def _stats_body(emb_ref, sum_ref, sq_ref):
    i = pl.program_id(0)
    x = emb_ref[...]
    s = jnp.sum(x, axis=1)
    q = jnp.sum(x * x, axis=1)

    @pl.when(i == 0)
    def _():
        sum_ref[...] = s
        sq_ref[...] = q

    @pl.when(i > 0)
    def _():
        sum_ref[...] += s
        sq_ref[...] += q


def _loss_body(emb_ref, rad_ref, rel_ref, sum_ref, sq_ref, gamma_ref,
               beta_ref, out_ref, *, batch):
    i = pl.program_id(0)
    inv_b = jnp.float32(1.0 / batch)
    s64 = sum_ref[:, :D] + sum_ref[:, D:]
    q64 = sq_ref[:, :D] + sq_ref[:, D:]
    mean = s64 * inv_b
    var = q64 * inv_b - mean * mean
    scale = gamma_ref[:, :D] * lax.rsqrt(var + BN_EPS)
    shift = beta_ref[:, :D] - mean * scale
    scale2 = jnp.concatenate([scale, scale], axis=-1)
    shift2 = jnp.concatenate([shift, shift], axis=-1)

    x = emb_ref[...]
    h = x * scale2[:, None, :] + shift2[:, None, :]
    r = jnp.abs(rad_ref[...])[:, 0]
    rel = rel_ref[...]

    lane = lax.broadcasted_iota(jnp.int32, (HP, 2 * D), 1)
    mlo = lane < D

    def norms(v):
        d2 = v * v
        qlo = jnp.sum(jnp.where(mlo, d2, 0.0), axis=-1)
        qhi = jnp.sum(jnp.where(mlo, 0.0, d2), axis=-1)
        tiny = jnp.float32(1e-12)
        return (jnp.sqrt(qlo + tiny).reshape(2, D * 2),
                jnp.sqrt(qhi + tiny).reshape(2, D * 2))

    relu = jax.nn.relu
    m = jnp.float32(MARGIN)

    def rlo(g):
        return r[g, 0:2, :]

    def rhi(g):
        return r[g, 2:4, :]

    total = jnp.float32(0.0)
    nlo, nhi = norms(h[0] - h[1])
    total += jnp.sum(relu(nlo + rlo(0) - rlo(1) - m))
    total += jnp.sum(relu(nhi + rhi(0) - rhi(1) - m))
    nlo, nhi = norms(h[3] - h[2])
    total += jnp.sum(relu(nlo - (rlo(2) + rlo(3)) - m))
    total += jnp.sum(relu(nhi - (rhi(2) + rhi(3)) - m))
    nlo, nhi = norms(h[4] - h[2])
    total += jnp.sum(relu(nlo - rlo(2) - m))
    total += jnp.sum(relu(nhi - rhi(2) - m))
    nlo, nhi = norms(h[4] - h[3])
    total += jnp.sum(relu(nlo - rlo(3) - m))
    total += jnp.sum(relu(nhi - rhi(3) - m))
    nlo, nhi = norms(h[5] + rel[0] - h[6])
    total += jnp.sum(relu(nlo + rlo(5) - rlo(6) - m))
    total += jnp.sum(relu(nhi + rhi(5) - rhi(6) - m))
    total += jnp.sum(relu(rlo(5) + rlo(6) - nlo + m))
    total += jnp.sum(relu(rhi(5) + rhi(6) - nhi + m))
    nlo, nhi = norms(h[7] - rel[1] - h[8])
    total += jnp.sum(relu(nlo - rlo(7) - rlo(8) - m))
    total += jnp.sum(relu(nhi - rhi(7) - rhi(8) - m))

    total = total.reshape(1, 1)

    @pl.when(i == 0)
    def _():
        out_ref[...] = total

    @pl.when(i > 0)
    def _():
        out_ref[...] += total

    @pl.when(i == pl.num_programs(0) - 1)
    def _():
        out_ref[...] *= inv_b


def kernel(gci0, gci1, gci2, gci3, class_embed, class_rad, rel_embed,
           bn_gamma, bn_beta):
    b = gci0.shape[0]
    cols = jnp.stack(
        [gci0[:, 0], gci0[:, 1],
         gci1[:, 0], gci1[:, 1], gci1[:, 2],
         gci2[:, 0], gci2[:, 2],
         gci3[:, 1], gci3[:, 2]], axis=0)
    relcols = jnp.stack([gci2[:, 1], gci3[:, 0]], axis=0)
    idx_nat = cols.reshape(NCOLS, b // CH, CH)
    idx_rel = relcols.reshape(NRELCOLS, b // CH, CH)

    emb4, rad_all, rel4 = _sc_gather(
        idx_nat, idx_rel, class_embed, class_rad, rel_embed, b)

    rad3 = rad_all.reshape(NCOLS, b // CH, 4, 2 * D)
    g2 = jnp.concatenate([bn_gamma, bn_gamma]).reshape(1, 2 * D)
    b2 = jnp.concatenate([bn_beta, bn_beta]).reshape(1, 2 * D)

    ng = b // CH

    sums, sqs = pl.pallas_call(
        _stats_body,
        grid=(ng,),
        in_specs=[pl.BlockSpec((NCOLS, HP, 2 * D), lambda i: (0, i, 0))],
        out_specs=(pl.BlockSpec((NCOLS, 2 * D), lambda i: (0, 0)),
                   pl.BlockSpec((NCOLS, 2 * D), lambda i: (0, 0))),
        out_shape=(jax.ShapeDtypeStruct((NCOLS, 2 * D), jnp.float32),
                   jax.ShapeDtypeStruct((NCOLS, 2 * D), jnp.float32)),
    )(emb4)

    loss = pl.pallas_call(
        functools.partial(_loss_body, batch=b),
        grid=(ng,),
        in_specs=[
            pl.BlockSpec((NCOLS, HP, 2 * D), lambda i: (0, i, 0)),
            pl.BlockSpec((NCOLS, 1, 4, 2 * D), lambda i: (0, i, 0, 0)),
            pl.BlockSpec((NRELCOLS, HP, 2 * D), lambda i: (0, i, 0)),
            pl.BlockSpec((NCOLS, 2 * D), lambda i: (0, 0)),
            pl.BlockSpec((NCOLS, 2 * D), lambda i: (0, 0)),
            pl.BlockSpec((1, 2 * D), lambda i: (0, 0)),
            pl.BlockSpec((1, 2 * D), lambda i: (0, 0)),
        ],
        out_specs=pl.BlockSpec((1, 1), lambda i: (0, 0)),
        out_shape=jax.ShapeDtypeStruct((1, 1), jnp.float32),
    )(emb4, rad3, rel4, sums, sqs, g2, b2)

    return loss[0, 0]
